# Initial kernel scaffold; baseline (speedup 1.0000x reference)
import jax
import jax.numpy as jnp
from jax.experimental import pallas as pl


def kernel(tertiary, sequence, subgraph_indices, knn_noise, W_in, Wq, Wk, Wv, Wo, M1, b1, M2, b2, M3, b3, Cw, Cb, Ew, Eb):
    raise NotImplementedError("write your pallas kernel here")


# speedup vs baseline: 4.9266x; 4.9266x over previous
import jax
import jax.numpy as jnp
from jax.experimental import pallas as pl

B, L, K = 8, 512, 15
N = B * L
SIZE = 128
HEADS = 8
DH = 16
HIDDEN = 512
DEPTH = 3
FEAT_IN = 27
MSG = SIZE + 29
MAXD = 20.0
KERNELS = 16


def _norm3(v):
    return jnp.sqrt(jnp.sum(v * v, axis=1, keepdims=True))


def _cross(a, b):
    ax, ay, az = a[:, 0:1], a[:, 1:2], a[:, 2:3]
    bx, by, bz = b[:, 0:1], b[:, 1:2], b[:, 2:3]
    return jnp.concatenate(
        [ay * bz - az * by, az * bx - ax * bz, ax * by - ay * bx], axis=1)


def _orient_body(pos_ref, o_ref):
    pos = pos_ref[...]
    d = pos[1:] - pos[:-1]
    d = d / (_norm3(d) + 1e-6)
    v1 = jnp.concatenate([d[0:1], d], axis=0)
    v2 = jnp.concatenate([d, d[N - 2:N - 1]], axis=0)
    b = v1 - v2
    b = b / (_norm3(b) + 1e-6)
    n = _cross(v1, v2)
    n = n / (_norm3(n) + 1e-6)
    c = _cross(b, n)
    o_ref[...] = jnp.concatenate([b, n, c], axis=1)


def _dot(a, b):
    return jnp.dot(a, b, preferred_element_type=jnp.float32)


def _main_body(pos_ref, post_ref, o_ref, noise_ref, win_ref, wq_ref, wk_ref,
               wv_ref, wo_ref, m1_ref, b1_ref, m2_ref, b2_ref, m3_ref, b3_ref,
               cwt_ref, cb_ref, ew_ref, eb_ref, out_ref):
    f32 = jnp.float32
    pos = pos_ref[...]
    post = post_ref[...]
    omat = o_ref[...]
    noise = noise_ref[0]

    px, py, pz = pos[:, 0:1], pos[:, 1:2], pos[:, 2:3]
    dx = px - post[0:1, :]
    dy = py - post[1:2, :]
    dz = pz - post[2:3, :]
    d2 = dx * dx + dy * dy + dz * dz
    closeness = -jnp.sqrt(d2 + 1e-8) + 3.0 * noise

    lane = jax.lax.broadcasted_iota(jnp.int32, (L, L), 1)
    idxs = []
    for _ in range(K):
        m = jnp.max(closeness, axis=1, keepdims=True)
        cand = jnp.where(closeness == m, lane, L)
        idx_k = jnp.min(cand, axis=1, keepdims=True)
        idxs.append(idx_k)
        closeness = jnp.where(lane == idx_k, -jnp.inf, closeness)

    pog = jnp.concatenate([pos, omat], axis=1)
    node = jax.lax.broadcasted_iota(jnp.int32, (L, 1), 0)
    mu = jax.lax.broadcasted_iota(jnp.int32, (1, KERNELS), 1).astype(f32) * (
        MAXD / (KERNELS - 1))
    sigma = MAXD / KERNELS

    ohs = []
    relfs = []
    for k in range(K):
        oh = (lane == idxs[k]).astype(f32)
        ohs.append(oh)
        g = _dot(oh, pog)
        rx = g[:, 0:1] - px
        ry = g[:, 1:2] - py
        rz = g[:, 2:3] - pz
        dist = jnp.sqrt(rx * rx + ry * ry + rz * rz + 1e-8)
        inv = 1.0 / (dist + 1e-6)
        t = (dist - mu) / sigma
        rbf = jnp.exp(-(t * t))
        rel_or = []
        for bb in range(3):
            for cc in range(3):
                acc = None
                for aa in range(3):
                    term = omat[:, 3 * aa + bb:3 * aa + bb + 1] * \
                        g[:, 3 + 3 * aa + cc:4 + 3 * aa + cc]
                    acc = term if acc is None else acc + term
                rel_or.append(acc)
        rel_ind = (idxs[k].astype(f32) - node.astype(f32)) * (1.0 / L)
        relf = jnp.concatenate(
            [rx * inv, ry * inv, rz * inv, rbf] + rel_or + [rel_ind], axis=1)
        relfs.append(relf)

    hl = jax.lax.broadcasted_iota(jnp.int32, (SIZE, HEADS), 0)
    hh = jax.lax.broadcasted_iota(jnp.int32, (SIZE, HEADS), 1)
    smat = ((hl // DH) == hh).astype(f32)
    st =((jax.lax.broadcasted_iota(jnp.int32, (HEADS, SIZE), 1) // DH) ==
          jax.lax.broadcasted_iota(jnp.int32, (HEADS, SIZE), 0)).astype(f32)

    x = jnp.sum(win_ref[...], axis=0, keepdims=True) + jnp.zeros((L, SIZE), f32)
    inv_sqrt_dh = 1.0 / (DH ** 0.5)
    for l in range(DEPTH):
        q = _dot(x, wq_ref[l])
        wkx, wkr = wk_ref[l, 0:SIZE, :], wk_ref[l, SIZE:MSG, :]
        wvx, wvr = wv_ref[l, 0:SIZE, :], wv_ref[l, SIZE:MSG, :]
        logits = []
        vvs = []
        for k in range(K):
            xg = _dot(ohs[k], x)
            kk = _dot(xg, wkx) + _dot(relfs[k], wkr)
            vv = _dot(xg, wvx) + _dot(relfs[k], wvr)
            vvs.append(vv)
            logits.append(_dot(q * kk, smat) * inv_sqrt_dh)
        m = logits[0]
        for k in range(1, K):
            m = jnp.maximum(m, logits[k])
        es = [jnp.exp(lg - m) for lg in logits]
        den = es[0]
        for k in range(1, K):
            den = den + es[k]
        inv_den = 1.0 / den
        attn = jnp.zeros((L, SIZE), f32)
        for k in range(K):
            attn = attn + _dot(es[k] * inv_den, st) * vvs[k]
        x = x + _dot(attn, wo_ref[l])
        h = jnp.maximum(_dot(x, m1_ref[l]) + b1_ref[l:l + 1, :], 0.0)
        h = jnp.maximum(_dot(h, m2_ref[l]) + b2_ref[l:l + 1, :], 0.0)
        x = x + _dot(h, m3_ref[l]) + b3_ref[l:l + 1, :]

    out = x
    cur = L
    for i in range(4):
        pad = jnp.zeros((1, SIZE), f32)
        xp = jnp.concatenate([pad, out, pad], axis=0)
        c = _dot(xp[0:cur], cwt_ref[3 * i]) + \
            _dot(xp[1:cur + 1], cwt_ref[3 * i + 1]) + \
            _dot(xp[2:cur + 2], cwt_ref[3 * i + 2])
        c = c + cb_ref[i:i + 1, :]
        c = jnp.where(c > 0, c, 0.01 * c)
        out = out + c
        half = cur // 2
        rr = jax.lax.broadcasted_iota(jnp.int32, (half, cur), 0)
        cc = jax.lax.broadcasted_iota(jnp.int32, (half, cur), 1)
        ee = (cc == 2 * rr).astype(f32)
        eo = (cc == 2 * rr + 1).astype(f32)
        out = jnp.maximum(_dot(ee, out), _dot(eo, out))
        cur = half

    pooled = jnp.sum(out, axis=0, keepdims=True)
    e = _dot(pooled, ew_ref[...]) + eb_ref[...]
    out_ref[0] = e * jnp.ones((1, SIZE), f32)


def kernel(tertiary, sequence, subgraph_indices, knn_noise, W_in, Wq, Wk, Wv,
           Wo, M1, b1, M2, b2, M3, b3, Cw, Cb, Ew, Eb):
    del sequence, subgraph_indices
    pos = tertiary[:, 1]

    omat = pl.pallas_call(
        _orient_body,
        out_shape=jax.ShapeDtypeStruct((N, 9), jnp.float32),
    )(pos)

    pos_t = pos.T
    cwt = jnp.transpose(Cw, (0, 3, 2, 1)).reshape(12, SIZE, SIZE)
    eb2 = Eb.reshape(1, 1)

    full = lambda *shape: pl.BlockSpec(shape, lambda b: (0,) * len(shape))
    out128 = pl.pallas_call(
        _main_body,
        grid=(B,),
        in_specs=[
            pl.BlockSpec((L, 3), lambda b: (b, 0)),
            pl.BlockSpec((3, L), lambda b: (0, b)),
            pl.BlockSpec((L, 9), lambda b: (b, 0)),
            pl.BlockSpec((1, L, L), lambda b: (b, 0, 0)),
            full(FEAT_IN, SIZE),
            full(DEPTH, SIZE, HEADS * DH),
            full(DEPTH, MSG, HEADS * DH),
            full(DEPTH, MSG, HEADS * DH),
            full(DEPTH, HEADS * DH, SIZE),
            full(DEPTH, SIZE, HIDDEN),
            full(DEPTH, HIDDEN),
            full(DEPTH, HIDDEN, HIDDEN),
            full(DEPTH, HIDDEN),
            full(DEPTH, HIDDEN, SIZE),
            full(DEPTH, SIZE),
            full(12, SIZE, SIZE),
            full(4, SIZE),
            full(SIZE, 1),
            full(1, 1),
        ],
        out_specs=pl.BlockSpec((1, 1, SIZE), lambda b: (b, 0, 0)),
        out_shape=jax.ShapeDtypeStruct((B, 1, SIZE), jnp.float32),
    )(pos, pos_t, omat, knn_noise, W_in, Wq, Wk, Wv, Wo, M1, b1, M2, b2, M3,
      b3, cwt, Cb, Ew, eb2)

    return out128[:, 0, 0:1]
